# Initial kernel scaffold; baseline (speedup 1.0000x reference)
import jax
import jax.numpy as jnp
from jax.experimental import pallas as pl


def kernel(pixel_feats, index_map):
    raise NotImplementedError("write your pallas kernel here")


# speedup vs baseline: 1.1034x; 1.1034x over previous
import functools

import jax
import jax.numpy as jnp
from jax import lax
from jax.experimental import pallas as pl
from jax.experimental.pallas import tpu as pltpu, tpu_sc as plsc

N_SPIXELS = 2304
NC, NS, L = 2, 16, 16
NW = NC * NS
HW = 384 * 384
NPLANES = 4 * 96
PPW = NPLANES // NW
CH = 3072
NCHUNK = HW // CH


def _body(data_hbm, idx_hbm, out_hbm, idx_v, data_v, accum, counts):
    wid = lax.axis_index("s") * NC + lax.axis_index("c")
    base_plane = wid * PPW

    zeros = jnp.zeros((L,), jnp.float32)
    ones = jnp.ones((L,), jnp.float32)

    def zero_acc(i, _):
        accum[pl.ds(i * L, L)] = zeros
        return 0
    lax.fori_loop(0, (PPW * N_SPIXELS) // L, zero_acc, 0)

    def zero_cnt(i, _):
        counts[pl.ds(i * L, L)] = zeros
        return 0
    lax.fori_loop(0, N_SPIXELS // L, zero_cnt, 0)

    def chunk_body(c, _):
        off = c * CH
        pltpu.sync_copy(idx_hbm.at[pl.ds(off, CH)], idx_v)
        for p in range(PPW):
            pltpu.sync_copy(data_hbm.at[base_plane + p, pl.ds(off, CH)],
                            data_v.at[p])

        def group_body(g, _):
            s = g * L
            iv = idx_v[pl.ds(s, L)]
            plsc.addupdate_scatter(counts, [iv], ones)
            for p in range(PPW):
                x = data_v[p, pl.ds(s, L)]
                plsc.addupdate_scatter(accum, [iv + (p * N_SPIXELS)], x)
            return 0
        lax.fori_loop(0, CH // L, group_body, 0)
        return 0
    lax.fori_loop(0, NCHUNK, chunk_body, 0)

    def fin_body(g, _):
        s = g * L
        inv = 1.0 / jnp.maximum(counts[pl.ds(s, L)], 1.0)
        for p in range(PPW):
            accum[pl.ds(p * N_SPIXELS + s, L)] = (
                accum[pl.ds(p * N_SPIXELS + s, L)] * inv)
        return 0
    lax.fori_loop(0, N_SPIXELS // L, fin_body, 0)

    for p in range(PPW):
        pltpu.sync_copy(accum.at[pl.ds(p * N_SPIXELS, N_SPIXELS)],
                        out_hbm.at[base_plane + p])


@jax.jit
def _spixel_feats(data, idx):
    mesh = plsc.VectorSubcoreMesh(core_axis_name="c", subcore_axis_name="s",
                                  num_cores=NC, num_subcores=NS)
    fn = pl.kernel(
        _body,
        out_type=jax.ShapeDtypeStruct((NPLANES, N_SPIXELS), jnp.float32),
        mesh=mesh,
        compiler_params=pltpu.CompilerParams(needs_layout_passes=False),
        scratch_types=[
            pltpu.VMEM((CH,), jnp.int32),
            pltpu.VMEM((PPW, CH), jnp.float32),
            pltpu.VMEM((PPW * N_SPIXELS,), jnp.float32),
            pltpu.VMEM((N_SPIXELS,), jnp.float32),
        ],
    )
    return fn(data, idx)


def kernel(pixel_feats, index_map):
    B, C, H, W = pixel_feats.shape
    data = pixel_feats.reshape(B * C, H * W)
    idx = index_map.reshape(-1)
    out = _spixel_feats(data, idx)
    return out.reshape(B, C, N_SPIXELS)


# speedup vs baseline: 1.5549x; 1.4092x over previous
import jax
import jax.numpy as jnp
from jax import lax
from jax.experimental import pallas as pl
from jax.experimental.pallas import tpu as pltpu, tpu_sc as plsc

N_SPIXELS = 2304
NC, NS, L = 2, 16, 16
NW = NC * NS
HW = 384 * 384
NPLANES = 4 * 96
PPW = NPLANES // NW
CH = 2048
NCHUNK = HW // CH


def _body(data_hbm, idx_hbm, out_hbm,
          idx0, idx1, dat0, dat1, accum, counts, sem0, sem1):
    wid = lax.axis_index("s") * NC + lax.axis_index("c")
    bufs = ((idx0, dat0, sem0), (idx1, dat1, sem1))

    zeros = jnp.zeros((L,), jnp.float32)
    ones = jnp.ones((L,), jnp.float32)

    def issue(c, b):
        idx_v, dat_v, sem = bufs[b]
        off = c * CH
        pltpu.async_copy(idx_hbm.at[pl.ds(off, CH)], idx_v, sem)
        pltpu.async_copy(data_hbm.at[pl.ds(wid, 1), :, pl.ds(off, CH)],
                         dat_v, sem)

    def wait(b):
        idx_v, dat_v, sem = bufs[b]
        pltpu.make_async_copy(idx_hbm.at[pl.ds(0, CH)], idx_v, sem).wait()
        pltpu.make_async_copy(data_hbm.at[pl.ds(0, 1), :, pl.ds(0, CH)],
                              dat_v, sem).wait()

    def compute(b):
        idx_v, dat_v, _ = bufs[b]

        def group_body(g, _):
            s = g * L
            iv = idx_v[pl.ds(s, L)]
            plsc.addupdate_scatter(counts, [iv], ones)
            for p in range(PPW):
                x = dat_v[0, p, pl.ds(s, L)]
                plsc.addupdate_scatter(
                    accum.at[pl.ds(p * N_SPIXELS, N_SPIXELS)], [iv], x)
            return 0
        lax.fori_loop(0, CH // L, group_body, 0, unroll=4)

    def zero_acc(i, _):
        s = i * L
        for p in range(PPW):
            accum[pl.ds(p * N_SPIXELS + s, L)] = zeros
        counts[pl.ds(s, L)] = zeros
        return 0
    lax.fori_loop(0, N_SPIXELS // L, zero_acc, 0)

    issue(0, 0)
    issue(1, 1)

    def pair_body(h, _):
        c0 = 2 * h
        wait(0)
        compute(0)

        @pl.when(c0 + 2 < NCHUNK)
        def _():
            issue(c0 + 2, 0)

        wait(1)
        compute(1)

        @pl.when(c0 + 3 < NCHUNK)
        def _():
            issue(c0 + 3, 1)
        return 0
    lax.fori_loop(0, NCHUNK // 2, pair_body, 0)

    def fin_body(g, _):
        s = g * L
        inv = 1.0 / jnp.maximum(counts[pl.ds(s, L)], 1.0)
        for p in range(PPW):
            accum[pl.ds(p * N_SPIXELS + s, L)] = (
                accum[pl.ds(p * N_SPIXELS + s, L)] * inv)
        return 0
    lax.fori_loop(0, N_SPIXELS // L, fin_body, 0)

    for p in range(PPW):
        pltpu.sync_copy(accum.at[pl.ds(p * N_SPIXELS, N_SPIXELS)],
                        out_hbm.at[wid * PPW + p])


@jax.jit
def _spixel_feats(data, idx):
    mesh = plsc.VectorSubcoreMesh(core_axis_name="c", subcore_axis_name="s",
                                  num_cores=NC, num_subcores=NS)
    fn = pl.kernel(
        _body,
        out_type=jax.ShapeDtypeStruct((NPLANES, N_SPIXELS), jnp.float32),
        mesh=mesh,
        compiler_params=pltpu.CompilerParams(needs_layout_passes=False),
        scratch_types=[
            pltpu.VMEM((CH,), jnp.int32),
            pltpu.VMEM((CH,), jnp.int32),
            pltpu.VMEM((1, PPW, CH), jnp.float32),
            pltpu.VMEM((1, PPW, CH), jnp.float32),
            pltpu.VMEM((PPW * N_SPIXELS,), jnp.float32),
            pltpu.VMEM((N_SPIXELS,), jnp.float32),
            pltpu.SemaphoreType.DMA,
            pltpu.SemaphoreType.DMA,
        ],
    )
    return fn(data, idx)


def kernel(pixel_feats, index_map):
    B, C, H, W = pixel_feats.shape
    data = pixel_feats.reshape(NW, PPW, H * W)
    idx = index_map.reshape(-1)
    out = _spixel_feats(data, idx)
    return out.reshape(B, C, N_SPIXELS)


# speedup vs baseline: 1.5581x; 1.0021x over previous
import jax
import jax.numpy as jnp
from jax import lax
from jax.experimental import pallas as pl
from jax.experimental.pallas import tpu as pltpu, tpu_sc as plsc

N_SPIXELS = 2304
NC, NS, L = 2, 16, 16
NW = NC * NS
HW = 384 * 384
NPLANES = 4 * 96
PPW = NPLANES // NW
CH = 2048
NCHUNK = HW // CH


def _body(data_hbm, idx_hbm, out_hbm,
          idx0, idx1, dat0, dat1, accums, counts, sem0, sem1):
    wid = lax.axis_index("s") * NC + lax.axis_index("c")
    bufs = ((idx0, dat0, sem0), (idx1, dat1, sem1))

    zeros = jnp.zeros((L,), jnp.float32)
    ones = jnp.ones((L,), jnp.float32)

    def issue(c, b):
        idx_v, dat_v, sem = bufs[b]
        off = c * CH
        pltpu.async_copy(idx_hbm.at[pl.ds(off, CH)], idx_v, sem)
        pltpu.async_copy(data_hbm.at[pl.ds(wid, 1), :, pl.ds(off, CH)],
                         dat_v, sem)

    def wait(b):
        idx_v, dat_v, sem = bufs[b]
        pltpu.make_async_copy(idx_hbm.at[pl.ds(0, CH)], idx_v, sem).wait()
        pltpu.make_async_copy(data_hbm.at[pl.ds(0, 1), :, pl.ds(0, CH)],
                              dat_v, sem).wait()

    def compute(b):
        idx_v, dat_v, _ = bufs[b]

        def group_body(g, _):
            s = g * L
            iv = idx_v[pl.ds(s, L)]
            plsc.addupdate_scatter(counts, [iv], ones)
            for p in range(PPW):
                x = dat_v[0, p, pl.ds(s, L)]
                plsc.addupdate_scatter(accums[p], [iv], x)
            return 0
        lax.fori_loop(0, CH // L, group_body, 0, unroll=8)

    def zero_acc(i, _):
        s = i * L
        for p in range(PPW):
            accums[p][pl.ds(s, L)] = zeros
        counts[pl.ds(s, L)] = zeros
        return 0
    lax.fori_loop(0, N_SPIXELS // L, zero_acc, 0)

    issue(0, 0)
    issue(1, 1)

    def pair_body(h, _):
        c0 = 2 * h
        wait(0)
        compute(0)

        @pl.when(c0 + 2 < NCHUNK)
        def _():
            issue(c0 + 2, 0)

        wait(1)
        compute(1)

        @pl.when(c0 + 3 < NCHUNK)
        def _():
            issue(c0 + 3, 1)
        return 0
    lax.fori_loop(0, NCHUNK // 2, pair_body, 0)

    def fin_body(g, _):
        s = g * L
        inv = 1.0 / jnp.maximum(counts[pl.ds(s, L)], 1.0)
        for p in range(PPW):
            accums[p][pl.ds(s, L)] = accums[p][pl.ds(s, L)] * inv
        return 0
    lax.fori_loop(0, N_SPIXELS // L, fin_body, 0)

    for p in range(PPW):
        pltpu.sync_copy(accums[p], out_hbm.at[wid * PPW + p])


@jax.jit
def _spixel_feats(data, idx):
    mesh = plsc.VectorSubcoreMesh(core_axis_name="c", subcore_axis_name="s",
                                  num_cores=NC, num_subcores=NS)
    fn = pl.kernel(
        _body,
        out_type=jax.ShapeDtypeStruct((NPLANES, N_SPIXELS), jnp.float32),
        mesh=mesh,
        compiler_params=pltpu.CompilerParams(needs_layout_passes=False),
        scratch_types=[
            pltpu.VMEM((CH,), jnp.int32),
            pltpu.VMEM((CH,), jnp.int32),
            pltpu.VMEM((1, PPW, CH), jnp.float32),
            pltpu.VMEM((1, PPW, CH), jnp.float32),
            [pltpu.VMEM((N_SPIXELS,), jnp.float32) for _ in range(PPW)],
            pltpu.VMEM((N_SPIXELS,), jnp.float32),
            pltpu.SemaphoreType.DMA,
            pltpu.SemaphoreType.DMA,
        ],
    )
    return fn(data, idx)


def kernel(pixel_feats, index_map):
    B, C, H, W = pixel_feats.shape
    data = pixel_feats.reshape(NW, PPW, H * W)
    idx = index_map.reshape(-1)
    out = _spixel_feats(data, idx)
    return out.reshape(B, C, N_SPIXELS)
